# baseline (speedup 1.0000x reference)
import functools

import jax
import jax.numpy as jnp
from jax import lax
from jax.experimental import pallas as pl
from jax.experimental.pallas import tpu as pltpu
from jax.experimental.pallas import tpu_sc as plsc

N = 10000
H = 128
BN = 32
NPAD = 10240
NC, NS = 2, 16
NW = NC * NS
G = 6
CHUNK = G * 128
NCHUNKS = 14
EPW = CHUNK * NCHUNKS
EPAD = NW * EPW
ROWS_PW = EPW // 128
DUMP = N
RPT = NPAD // NS
RB = 1024


def _sc_mesh():
    return plsc.VectorSubcoreMesh(
        core_axis_name="c", subcore_axis_name="s", num_cores=NC, num_subcores=NS
    )


_SC_PARAMS = pltpu.CompilerParams(
    use_tc_tiling_on_sc=False, needs_layout_passes=False
)


def _vrsqrt(x):
    xh = x * 0.5
    i = plsc.bitcast(x, jnp.int32)
    i = jnp.full((16,), 0x5F3759DF, jnp.int32) - (i >> 1)
    y = plsc.bitcast(i, jnp.float32)
    for _ in range(3):
        y = y * (1.5 - xh * y * y)
    return y


def _deg_pass(dst2d, ones_c, z16):
    @functools.partial(
        pl.kernel,
        out_type=jax.ShapeDtypeStruct((NC, NPAD, 16), jnp.float32),
        mesh=_sc_mesh(),
        scratch_types=[
            pltpu.VMEM((ROWS_PW, 1, 128), jnp.int32),
            pltpu.VMEM((128, 16), jnp.float32),
            pltpu.VMEM_SHARED((NPAD, 16), jnp.float32),
            pltpu.SemaphoreType.DMA,
        ],
        compiler_params=_SC_PARAMS,
    )
    def deg_kernel(dst_hbm, ones_hbm, z_hbm, out_hbm, dstv, ones_v, acc, sem):
        c = lax.axis_index("c")
        s = lax.axis_index("s")
        w = c * NS + s
        pltpu.sync_copy(z_hbm, acc.at[pl.ds(s * RPT, RPT)])
        pltpu.sync_copy(ones_hbm, ones_v)
        pltpu.sync_copy(dst_hbm.at[pl.ds(w * ROWS_PW, ROWS_PW)], dstv)
        plsc.subcore_barrier()
        descs = []
        for r in range(ROWS_PW):
            if r >= 12:
                descs[r - 12].wait()
            descs.append(
                pltpu.async_copy(ones_v, acc.at[dstv.at[r, 0]], sem, add=True)
            )
        for d in descs[-12:]:
            d.wait()
        plsc.subcore_barrier()
        pltpu.sync_copy(
            acc.at[pl.ds(s * RPT, RPT)], out_hbm.at[c, pl.ds(s * RPT, RPT)]
        )

    return deg_kernel(dst2d, ones_c, z16)


def _edge_loop(tbl, srcv, dstv, rows, gsem, ssem, acc):
    def fire_gather(k):
        b = k & 1
        return [
            pltpu.async_copy(
                tbl.at[srcv.at[k * G + g, 0]],
                rows[b].at[pl.ds(g * 128, 128)],
                gsem[b],
            )
            for g in range(G)
        ]

    def fire_scatter(k):
        b = k & 1
        return [
            pltpu.async_copy(
                rows[b].at[pl.ds(g * 128, 128)],
                acc.at[dstv.at[k * G + g, 0]],
                ssem[b],
                add=True,
            )
            for g in range(G)
        ]

    gd = {0: fire_gather(0)}
    sd = {}
    for k in range(NCHUNKS):
        for d in gd[k]:
            d.wait()
        sd[k] = fire_scatter(k)
        if k + 1 < NCHUNKS:
            if k - 1 >= 0:
                for d in sd[k - 1]:
                    d.wait()
            gd[k + 1] = fire_gather(k + 1)
    for k in (NCHUNKS - 2, NCHUNKS - 1):
        for d in sd[k]:
            d.wait()


def _conv1_pass(h0, degp, src2d, dst2d, z32):
    @functools.partial(
        pl.kernel,
        out_type=(
            jax.ShapeDtypeStruct((NC, NPAD, 2, 16), jnp.float32),
            jax.ShapeDtypeStruct((NPAD, 16), jnp.float32),
        ),
        mesh=_sc_mesh(),
        scratch_types=[
            pltpu.VMEM((ROWS_PW, 1, 128), jnp.int32),
            pltpu.VMEM((ROWS_PW, 1, 128), jnp.int32),
            pltpu.VMEM((CHUNK, 2, 16), jnp.float32),
            pltpu.VMEM((CHUNK, 2, 16), jnp.float32),
            pltpu.VMEM((RPT, 16), jnp.float32),
            pltpu.SemaphoreType.DMA,
            pltpu.SemaphoreType.DMA,
            pltpu.SemaphoreType.DMA,
            pltpu.SemaphoreType.DMA,
            pltpu.VMEM_SHARED((NPAD, 2, 16), jnp.float32),
            pltpu.HBM((NPAD, 2, 16), jnp.float32),
        ],
        compiler_params=_SC_PARAMS,
    )
    def conv1_kernel(h0_hbm, degp_hbm, src_hbm, dst_hbm, z_hbm,
                     out_hbm, dis_hbm,
                     srcv, dstv, rows0, rows1, disv,
                     gsem0, gsem1, ssem0, ssem1, acc, tbl):
        c = lax.axis_index("c")
        s = lax.axis_index("s")
        w = c * NS + s
        r0 = s * RPT
        hp = RPT // 2
        pltpu.sync_copy(z_hbm, acc.at[pl.ds(r0, RPT)])
        pltpu.sync_copy(src_hbm.at[pl.ds(w * ROWS_PW, ROWS_PW)], srcv)
        pltpu.sync_copy(dst_hbm.at[pl.ds(w * ROWS_PW, ROWS_PW)], dstv)
        pltpu.sync_copy(degp_hbm.at[0, pl.ds(s * hp, hp)], rows1.at[pl.ds(0, hp)])
        pltpu.sync_copy(degp_hbm.at[1, pl.ds(s * hp, hp)], rows1.at[pl.ds(hp, hp)])
        pltpu.sync_copy(h0_hbm.at[pl.ds(r0, RPT)], rows0.at[pl.ds(0, RPT)])

        def pro(p, carry):
            for h in range(2):
                y = _vrsqrt(jnp.maximum(rows1[p, h] + rows1[hp + p, h], 1.0))
                disv[2 * p + h] = y
                rows0[2 * p + h, 0] = rows0[2 * p + h, 0] * y
                rows0[2 * p + h, 1] = rows0[2 * p + h, 1] * y
            return carry

        lax.fori_loop(0, hp, pro, 0)
        pltpu.sync_copy(rows0.at[pl.ds(0, RPT)], tbl.at[pl.ds(r0, RPT)])
        pltpu.sync_copy(disv, dis_hbm.at[pl.ds(r0, RPT)])
        plsc.subcore_barrier()

        _edge_loop(tbl, srcv, dstv, (rows0, rows1),
                   (gsem0, gsem1), (ssem0, ssem1), acc)
        plsc.subcore_barrier()
        pltpu.sync_copy(
            acc.at[pl.ds(r0, RPT)], out_hbm.at[c, pl.ds(r0, RPT)]
        )

    return conv1_kernel(h0, degp, src2d, dst2d, z32)


def _conv2_pass(m1p, dis16, b2x16, src2d, dst2d, z32):
    @functools.partial(
        pl.kernel,
        out_type=jax.ShapeDtypeStruct((NC, NPAD, 2, 16), jnp.float32),
        mesh=_sc_mesh(),
        scratch_types=[
            pltpu.VMEM((ROWS_PW, 1, 128), jnp.int32),
            pltpu.VMEM((ROWS_PW, 1, 128), jnp.int32),
            pltpu.VMEM((CHUNK, 2, 16), jnp.float32),
            pltpu.VMEM((CHUNK, 2, 16), jnp.float32),
            pltpu.VMEM((RPT, 16), jnp.float32),
            pltpu.VMEM((2, 16), jnp.float32),
            pltpu.SemaphoreType.DMA,
            pltpu.SemaphoreType.DMA,
            pltpu.SemaphoreType.DMA,
            pltpu.SemaphoreType.DMA,
            pltpu.VMEM_SHARED((NPAD, 2, 16), jnp.float32),
            pltpu.HBM((NPAD, 2, 16), jnp.float32),
        ],
        compiler_params=_SC_PARAMS,
    )
    def conv2_kernel(m1p_hbm, dis_hbm, b_hbm, src_hbm, dst_hbm, z_hbm,
                     out_hbm,
                     srcv, dstv, rows0, rows1, disv, bv,
                     gsem0, gsem1, ssem0, ssem1, acc, tbl):
        c = lax.axis_index("c")
        s = lax.axis_index("s")
        w = c * NS + s
        r0 = s * RPT
        pltpu.sync_copy(z_hbm, acc.at[pl.ds(r0, RPT)])
        pltpu.sync_copy(src_hbm.at[pl.ds(w * ROWS_PW, ROWS_PW)], srcv)
        pltpu.sync_copy(dst_hbm.at[pl.ds(w * ROWS_PW, ROWS_PW)], dstv)
        pltpu.sync_copy(m1p_hbm.at[0, pl.ds(r0, RPT)], rows0.at[pl.ds(0, RPT)])
        pltpu.sync_copy(m1p_hbm.at[1, pl.ds(r0, RPT)], rows1.at[pl.ds(0, RPT)])
        pltpu.sync_copy(dis_hbm.at[pl.ds(r0, RPT)], disv)
        pltpu.sync_copy(b_hbm, bv)

        def pro(r, carry):
            y = disv[r]
            for h in range(2):
                v = (rows0[r, h] + rows1[r, h]) * y
                rows0[r, h] = jnp.maximum(v + bv[h], 0.0) * y
            return carry

        lax.fori_loop(0, RPT, pro, 0)
        pltpu.sync_copy(rows0.at[pl.ds(0, RPT)], tbl.at[pl.ds(r0, RPT)])
        plsc.subcore_barrier()

        _edge_loop(tbl, srcv, dstv, (rows0, rows1),
                   (gsem0, gsem1), (ssem0, ssem1), acc)
        plsc.subcore_barrier()
        pltpu.sync_copy(
            acc.at[pl.ds(r0, RPT)], out_hbm.at[c, pl.ds(r0, RPT)]
        )

    return conv2_kernel(m1p, dis16, b2x16, src2d, dst2d, z32)


def _tc_down(x_pad, w_down):
    def body(x_ref, wd_ref, h0_ref):
        h0_ref[...] = lax.dot_general(
            x_ref[...], wd_ref[...], (((1,), (1,)), ((), ())),
            preferred_element_type=jnp.float32,
        )

    return pl.pallas_call(
        body,
        grid=(NPAD // RB,),
        in_specs=[
            pl.BlockSpec((RB, H), lambda i: (i, 0)),
            pl.BlockSpec((BN, H), lambda i: (0, 0)),
        ],
        out_specs=pl.BlockSpec((RB, BN), lambda i: (i, 0)),
        out_shape=jax.ShapeDtypeStruct((NPAD, BN), jnp.float32),
    )(x_pad, w_down)


def _tc_up(m2p, dis16, w_up, b_up_row, x_pad):
    def body(m2p_ref, dis_ref, wu_ref, b_ref, x_ref, out_ref):
        m2 = (m2p_ref[0] + m2p_ref[1]) * dis_ref[...][:, :1]
        y = lax.dot_general(
            m2, wu_ref[...], (((1,), (1,)), ((), ())),
            preferred_element_type=jnp.float32,
        )
        out_ref[...] = y + b_ref[...] + x_ref[...]

    return pl.pallas_call(
        body,
        grid=(NPAD // RB,),
        in_specs=[
            pl.BlockSpec((NC, RB, BN), lambda i: (0, i, 0)),
            pl.BlockSpec((RB, 16), lambda i: (i, 0)),
            pl.BlockSpec((H, BN), lambda i: (0, 0)),
            pl.BlockSpec((1, H), lambda i: (0, 0)),
            pl.BlockSpec((RB, H), lambda i: (i, 0)),
        ],
        out_specs=pl.BlockSpec((RB, H), lambda i: (i, 0)),
        out_shape=jax.ShapeDtypeStruct((NPAD, H), jnp.float32),
    )(m2p, dis16, w_up, b_up_row, x_pad)


def kernel(x, edge_index, W_down, b_down, W_up, b_up):
    f32 = jnp.float32
    e = edge_index.shape[1]
    npadding = EPAD - e - N
    loop = jnp.arange(N, dtype=jnp.int32)
    fill = DUMP + (jnp.arange(npadding, dtype=jnp.int32) % (NPAD - N))
    src2d = jnp.concatenate([edge_index[0], loop, fill]).reshape(-1, 1, 128)
    dst2d = jnp.concatenate([edge_index[1], loop, fill]).reshape(-1, 1, 128)
    x_pad = jnp.pad(x, ((0, NPAD - N), (0, 0)))
    z16 = jnp.zeros((RPT, 16), f32)
    z32 = jnp.zeros((RPT, 2, 16), f32)
    ones_c = jnp.ones((128, 16), f32)

    h0 = _tc_down(x_pad, W_down).reshape(NPAD, 2, 16)
    degp = _deg_pass(dst2d, ones_c, z16)
    m1p, dis16 = _conv1_pass(
        h0, degp.reshape(NC, NPAD // 2, 2, 16), src2d, dst2d, z32
    )
    m2p = _conv2_pass(m1p, dis16, b_down.reshape(2, 16), src2d, dst2d, z32)
    out = _tc_up(m2p.reshape(NC, NPAD, BN), dis16, W_up,
                 b_up.reshape(1, H), x_pad)
    return out[:N]


# speedup vs baseline: 1.1193x; 1.1193x over previous
import functools

import jax
import jax.numpy as jnp
from jax import lax
from jax.experimental import pallas as pl
from jax.experimental.pallas import tpu as pltpu
from jax.experimental.pallas import tpu_sc as plsc

N = 10000
H = 128
BN = 32
NPAD = 10240
NC, NS = 2, 16
NW = NC * NS
G = 6
CHUNK = G * 128
NCHUNKS = 14
EPW = CHUNK * NCHUNKS
EPAD = NW * EPW
ROWS_PW = EPW // 128
DUMP = N
RPT = NPAD // NS
RB = 1024


def _sc_mesh():
    return plsc.VectorSubcoreMesh(
        core_axis_name="c", subcore_axis_name="s", num_cores=NC, num_subcores=NS
    )


_SC_PARAMS = pltpu.CompilerParams(
    use_tc_tiling_on_sc=False, needs_layout_passes=False
)


def _vrsqrt(x):
    xh = x * 0.5
    i = plsc.bitcast(x, jnp.int32)
    i = jnp.full((16,), 0x5F3759DF, jnp.int32) - (i >> 1)
    y = plsc.bitcast(i, jnp.float32)
    for _ in range(3):
        y = y * (1.5 - xh * y * y)
    return y


def _deg_pass(dst2d, ones_c, z16):
    @functools.partial(
        pl.kernel,
        out_type=jax.ShapeDtypeStruct((NC, NPAD, 16), jnp.float32),
        mesh=_sc_mesh(),
        scratch_types=[
            pltpu.VMEM((ROWS_PW, 1, 128), jnp.int32),
            pltpu.VMEM((128, 16), jnp.float32),
            pltpu.VMEM_SHARED((NPAD, 16), jnp.float32),
            pltpu.SemaphoreType.DMA,
        ],
        compiler_params=_SC_PARAMS,
    )
    def deg_kernel(dst_hbm, ones_hbm, z_hbm, out_hbm, dstv, ones_v, acc, sem):
        c = lax.axis_index("c")
        s = lax.axis_index("s")
        w = c * NS + s
        pltpu.sync_copy(z_hbm, acc.at[pl.ds(s * RPT, RPT)])
        pltpu.sync_copy(ones_hbm, ones_v)
        pltpu.sync_copy(dst_hbm.at[pl.ds(w * ROWS_PW, ROWS_PW)], dstv)
        plsc.subcore_barrier()
        descs = []
        for r in range(ROWS_PW):
            if r >= 12:
                descs[r - 12].wait()
            descs.append(
                pltpu.async_copy(ones_v, acc.at[dstv.at[r, 0]], sem, add=True)
            )
        for d in descs[-12:]:
            d.wait()
        plsc.subcore_barrier()
        pltpu.sync_copy(
            acc.at[pl.ds(s * RPT, RPT)], out_hbm.at[c, pl.ds(s * RPT, RPT)]
        )

    return deg_kernel(dst2d, ones_c, z16)


def _edge_loop(tbl, srcv, dstv, rows, gsem, ssem, acc):
    def fire_gather(k):
        b = k & 1
        return [
            pltpu.async_copy(
                tbl.at[srcv.at[k * G + g, 0]],
                rows[b].at[pl.ds(g * 128, 128)],
                gsem[b],
            )
            for g in range(G)
        ]

    def fire_scatter(k):
        b = k & 1
        return [
            pltpu.async_copy(
                rows[b].at[pl.ds(g * 128, 128)],
                acc.at[dstv.at[k * G + g, 0]],
                ssem[b],
                add=True,
            )
            for g in range(G)
        ]

    gd = {0: fire_gather(0)}
    sd = {}
    for k in range(NCHUNKS):
        for d in gd[k]:
            d.wait()
        sd[k] = fire_scatter(k)
        if k + 1 < NCHUNKS:
            if k - 1 >= 0:
                for d in sd[k - 1]:
                    d.wait()
            gd[k + 1] = fire_gather(k + 1)
    for k in (NCHUNKS - 2, NCHUNKS - 1):
        for d in sd[k]:
            d.wait()


def _conv1_pass(h0, degp, src2d, dst2d, z32):
    @functools.partial(
        pl.kernel,
        out_type=(
            jax.ShapeDtypeStruct((NC, NPAD, 2, 16), jnp.float32),
            jax.ShapeDtypeStruct((NPAD, 16), jnp.float32),
        ),
        mesh=_sc_mesh(),
        scratch_types=[
            pltpu.VMEM((ROWS_PW, 1, 128), jnp.int32),
            pltpu.VMEM((ROWS_PW, 1, 128), jnp.int32),
            pltpu.VMEM((CHUNK, 2, 16), jnp.float32),
            pltpu.VMEM((CHUNK, 2, 16), jnp.float32),
            pltpu.VMEM((RPT, 16), jnp.float32),
            pltpu.VMEM((RPT, 16), jnp.float32),
            pltpu.SemaphoreType.DMA,
            pltpu.SemaphoreType.DMA,
            pltpu.SemaphoreType.DMA,
            pltpu.SemaphoreType.DMA,
            pltpu.VMEM_SHARED((NPAD, 2, 16), jnp.float32),
            pltpu.HBM((NPAD, 2, 16), jnp.float32),
        ],
        compiler_params=_SC_PARAMS,
    )
    def conv1_kernel(h0_hbm, degp_hbm, src_hbm, dst_hbm, z_hbm,
                     out_hbm, dis_hbm,
                     srcv, dstv, rows0, rows1, dg0v, disv,
                     gsem0, gsem1, ssem0, ssem1, acc, tbl):
        c = lax.axis_index("c")
        s = lax.axis_index("s")
        w = c * NS + s
        r0 = s * RPT
        pltpu.sync_copy(z_hbm, acc.at[pl.ds(r0, RPT)])
        pltpu.sync_copy(src_hbm.at[pl.ds(w * ROWS_PW, ROWS_PW)], srcv)
        pltpu.sync_copy(dst_hbm.at[pl.ds(w * ROWS_PW, ROWS_PW)], dstv)
        pltpu.sync_copy(degp_hbm.at[0, pl.ds(r0, RPT)], dg0v)
        pltpu.sync_copy(degp_hbm.at[1, pl.ds(r0, RPT)], disv)
        pltpu.sync_copy(h0_hbm.at[pl.ds(r0, RPT)], rows0.at[pl.ds(0, RPT)])

        def pro(p, carry):
            for u in range(4):
                r = 4 * p + u
                y = _vrsqrt(jnp.maximum(dg0v[r] + disv[r], 1.0))
                disv[r] = y
                rows0[r, 0] = rows0[r, 0] * y
                rows0[r, 1] = rows0[r, 1] * y
            return carry

        lax.fori_loop(0, RPT // 4, pro, 0)
        pltpu.sync_copy(rows0.at[pl.ds(0, RPT)], tbl.at[pl.ds(r0, RPT)])
        pltpu.sync_copy(disv, dis_hbm.at[pl.ds(r0, RPT)])
        plsc.subcore_barrier()

        _edge_loop(tbl, srcv, dstv, (rows0, rows1),
                   (gsem0, gsem1), (ssem0, ssem1), acc)
        plsc.subcore_barrier()
        pltpu.sync_copy(
            acc.at[pl.ds(r0, RPT)], out_hbm.at[c, pl.ds(r0, RPT)]
        )

    return conv1_kernel(h0, degp, src2d, dst2d, z32)


def _conv2_pass(m1p, dis16, b2x16, src2d, dst2d, z32):
    @functools.partial(
        pl.kernel,
        out_type=jax.ShapeDtypeStruct((NC, NPAD, 2, 16), jnp.float32),
        mesh=_sc_mesh(),
        scratch_types=[
            pltpu.VMEM((ROWS_PW, 1, 128), jnp.int32),
            pltpu.VMEM((ROWS_PW, 1, 128), jnp.int32),
            pltpu.VMEM((CHUNK, 2, 16), jnp.float32),
            pltpu.VMEM((CHUNK, 2, 16), jnp.float32),
            pltpu.VMEM((RPT, 16), jnp.float32),
            pltpu.VMEM((2, 16), jnp.float32),
            pltpu.SemaphoreType.DMA,
            pltpu.SemaphoreType.DMA,
            pltpu.SemaphoreType.DMA,
            pltpu.SemaphoreType.DMA,
            pltpu.VMEM_SHARED((NPAD, 2, 16), jnp.float32),
            pltpu.HBM((NPAD, 2, 16), jnp.float32),
        ],
        compiler_params=_SC_PARAMS,
    )
    def conv2_kernel(m1p_hbm, dis_hbm, b_hbm, src_hbm, dst_hbm, z_hbm,
                     out_hbm,
                     srcv, dstv, rows0, rows1, disv, bv,
                     gsem0, gsem1, ssem0, ssem1, acc, tbl):
        c = lax.axis_index("c")
        s = lax.axis_index("s")
        w = c * NS + s
        r0 = s * RPT
        pltpu.sync_copy(z_hbm, acc.at[pl.ds(r0, RPT)])
        pltpu.sync_copy(src_hbm.at[pl.ds(w * ROWS_PW, ROWS_PW)], srcv)
        pltpu.sync_copy(dst_hbm.at[pl.ds(w * ROWS_PW, ROWS_PW)], dstv)
        pltpu.sync_copy(m1p_hbm.at[0, pl.ds(r0, RPT)], rows0.at[pl.ds(0, RPT)])
        pltpu.sync_copy(m1p_hbm.at[1, pl.ds(r0, RPT)], rows1.at[pl.ds(0, RPT)])
        pltpu.sync_copy(dis_hbm.at[pl.ds(r0, RPT)], disv)
        pltpu.sync_copy(b_hbm, bv)

        def pro(p, carry):
            for u in range(4):
                r = 4 * p + u
                y = disv[r]
                for h in range(2):
                    v = (rows0[r, h] + rows1[r, h]) * y
                    rows0[r, h] = jnp.maximum(v + bv[h], 0.0) * y
            return carry

        lax.fori_loop(0, RPT // 4, pro, 0)
        pltpu.sync_copy(rows0.at[pl.ds(0, RPT)], tbl.at[pl.ds(r0, RPT)])
        plsc.subcore_barrier()

        _edge_loop(tbl, srcv, dstv, (rows0, rows1),
                   (gsem0, gsem1), (ssem0, ssem1), acc)
        plsc.subcore_barrier()
        pltpu.sync_copy(
            acc.at[pl.ds(r0, RPT)], out_hbm.at[c, pl.ds(r0, RPT)]
        )

    return conv2_kernel(m1p, dis16, b2x16, src2d, dst2d, z32)


def _tc_down(x_pad, w_down):
    def body(x_ref, wd_ref, h0_ref):
        h0 = lax.dot_general(
            x_ref[...], wd_ref[...], (((1,), (1,)), ((), ())),
            preferred_element_type=jnp.float32,
        )
        h0_ref[...] = h0.reshape(RB, 2, 16)

    return pl.pallas_call(
        body,
        grid=(NPAD // RB,),
        in_specs=[
            pl.BlockSpec((RB, H), lambda i: (i, 0)),
            pl.BlockSpec((BN, H), lambda i: (0, 0)),
        ],
        out_specs=pl.BlockSpec((RB, 2, 16), lambda i: (i, 0, 0)),
        out_shape=jax.ShapeDtypeStruct((NPAD, 2, 16), jnp.float32),
    )(x_pad, w_down)


def _tc_up(m2p, dis16, w_up, b_up_row, x_pad):
    def body(m2p_ref, dis_ref, wu_ref, b_ref, x_ref, out_ref):
        m2 = ((m2p_ref[0] + m2p_ref[1]) * dis_ref[...][:, None, :]).reshape(
            RB, BN
        )
        y = lax.dot_general(
            m2, wu_ref[...], (((1,), (1,)), ((), ())),
            preferred_element_type=jnp.float32,
        )
        out_ref[...] = y + b_ref[...] + x_ref[...]

    return pl.pallas_call(
        body,
        grid=(NPAD // RB,),
        in_specs=[
            pl.BlockSpec((NC, RB, 2, 16), lambda i: (0, i, 0, 0)),
            pl.BlockSpec((RB, 16), lambda i: (i, 0)),
            pl.BlockSpec((H, BN), lambda i: (0, 0)),
            pl.BlockSpec((1, H), lambda i: (0, 0)),
            pl.BlockSpec((RB, H), lambda i: (i, 0)),
        ],
        out_specs=pl.BlockSpec((RB, H), lambda i: (i, 0)),
        out_shape=jax.ShapeDtypeStruct((NPAD, H), jnp.float32),
    )(m2p, dis16, w_up, b_up_row, x_pad)


def kernel(x, edge_index, W_down, b_down, W_up, b_up):
    f32 = jnp.float32
    e = edge_index.shape[1]
    npadding = EPAD - e - N
    loop = jnp.arange(N, dtype=jnp.int32)
    fill = DUMP + (jnp.arange(npadding, dtype=jnp.int32) % (NPAD - N))
    src2d = jnp.concatenate([edge_index[0], loop, fill]).reshape(-1, 1, 128)
    dst2d = jnp.concatenate([edge_index[1], loop, fill]).reshape(-1, 1, 128)
    x_pad = jnp.pad(x, ((0, NPAD - N), (0, 0)))
    z16 = jnp.zeros((RPT, 16), f32)
    z32 = jnp.zeros((RPT, 2, 16), f32)
    ones_c = jnp.ones((128, 16), f32)

    h0 = _tc_down(x_pad, W_down)
    degp = _deg_pass(dst2d, ones_c, z16)
    m1p, dis16 = _conv1_pass(h0, degp, src2d, dst2d, z32)
    m2p = _conv2_pass(m1p, dis16, b_down.reshape(2, 16), src2d, dst2d, z32)
    out = _tc_up(m2p, dis16, W_up, b_up.reshape(1, H), x_pad)
    return out[:N]


# speedup vs baseline: 1.8364x; 1.6406x over previous
import functools

import jax
import jax.numpy as jnp
from jax import lax
from jax.experimental import pallas as pl
from jax.experimental.pallas import tpu as pltpu
from jax.experimental.pallas import tpu_sc as plsc

N = 10000
H = 128
BN = 32
NPAD = 10240
NC, NS = 2, 16
NW = NC * NS
G = 6
DUMP = N
RPT = NPAD // NS
HPT = RPT // 2
RB = 1000


def _sc_mesh():
    return plsc.VectorSubcoreMesh(
        core_axis_name="c", subcore_axis_name="s", num_cores=NC, num_subcores=NS
    )


_SC_PARAMS = pltpu.CompilerParams(
    use_tc_tiling_on_sc=False, needs_layout_passes=False
)


def _vrsqrt(x):
    xh = x * 0.5
    i = plsc.bitcast(x, jnp.int32)
    i = jnp.full((16,), 0x5F3759DF, jnp.int32) - (i >> 1)
    y = plsc.bitcast(i, jnp.float32)
    for _ in range(3):
        y = y * (1.5 - xh * y * y)
    return y


def _edge_geometry(e_rows):
    base = e_rows // NW
    extra = e_rows % NW
    chunks = [G] * (base // G)
    if base % G:
        chunks.append(base % G)
    return base, extra, chunks


def _stage_edges(e_hbm, row, base, extra, e_rows, srcv, dstv, w):
    pltpu.sync_copy(e_hbm.at[0, pl.ds(w * base, base)], srcv.at[pl.ds(0, base)])
    pltpu.sync_copy(e_hbm.at[1, pl.ds(w * base, base)], dstv.at[pl.ds(0, base)])
    del row
    if extra:
        off = e_rows - extra + lax.min(w, extra - 1)
        pltpu.sync_copy(e_hbm.at[0, pl.ds(off, 1)], srcv.at[pl.ds(base, 1)])
        pltpu.sync_copy(e_hbm.at[1, pl.ds(off, 1)], dstv.at[pl.ds(base, 1)])


def _edge_loop(tbl, srcv, dstv, rows, gsem, ssem, acc, chunks):
    starts = [0]
    for g in chunks:
        starts.append(starts[-1] + g)
    nch = len(chunks)

    def fire_gather(k):
        b = k & 1
        return [
            pltpu.async_copy(
                tbl.at[srcv.at[starts[k] + g, 0]],
                rows[b].at[pl.ds(g * 128, 128)],
                gsem[b],
            )
            for g in range(chunks[k])
        ]

    def fire_scatter(k):
        b = k & 1
        return [
            pltpu.async_copy(
                rows[b].at[pl.ds(g * 128, 128)],
                acc.at[dstv.at[starts[k] + g, 0]],
                ssem[b],
                add=True,
            )
            for g in range(chunks[k])
        ]

    gd = {0: fire_gather(0)}
    sd = {}
    for k in range(nch):
        for d in gd[k]:
            d.wait()
        sd[k] = fire_scatter(k)
        if k + 1 < nch:
            if k - 1 >= 0:
                for d in sd[k - 1]:
                    d.wait()
            gd[k + 1] = fire_gather(k + 1)
    for k in range(max(0, nch - 2), nch):
        for d in sd[k]:
            d.wait()


def _extra_edge(tbl, srcv, dstv, buf, sem, acc, base, extra, w):
    if not extra:
        return

    @pl.when(w < extra)
    def _():
        pltpu.async_copy(
            tbl.at[srcv.at[base, 0]], buf.at[pl.ds(0, 128)], sem
        ).wait()
        pltpu.async_copy(
            buf.at[pl.ds(0, 128)], acc.at[dstv.at[base, 0]], sem, add=True
        ).wait()


def _deg_pass(e2, ones_r, z16, e_rows):
    base, extra, chunks = _edge_geometry(e_rows)

    @functools.partial(
        pl.kernel,
        out_type=jax.ShapeDtypeStruct((NC, NPAD, 16), jnp.float32),
        mesh=_sc_mesh(),
        scratch_types=[
            pltpu.VMEM((base + 1, 1, 128), jnp.int32),
            pltpu.VMEM((128, 16), jnp.float32),
            pltpu.VMEM_SHARED((NPAD, 16), jnp.float32),
            pltpu.SemaphoreType.DMA,
        ],
        compiler_params=_SC_PARAMS,
    )
    def deg_kernel(e_hbm, ones_hbm, z_hbm, out_hbm, dstv, ones_v, acc, sem):
        c = lax.axis_index("c")
        s = lax.axis_index("s")
        w = c * NS + s
        r0 = s * RPT

        @pl.when(c == 0)
        def _():
            pltpu.sync_copy(ones_hbm, acc.at[pl.ds(r0, RPT)])

        @pl.when(c != 0)
        def _():
            pltpu.sync_copy(z_hbm, acc.at[pl.ds(r0, RPT)])

        pltpu.sync_copy(ones_hbm.at[pl.ds(0, 128)], ones_v)
        pltpu.sync_copy(e_hbm.at[1, pl.ds(w * base, base)],
                        dstv.at[pl.ds(0, base)])
        if extra:
            off = e_rows - extra + lax.min(w, extra - 1)
            pltpu.sync_copy(e_hbm.at[1, pl.ds(off, 1)], dstv.at[pl.ds(base, 1)])
        plsc.subcore_barrier()
        if extra:
            @pl.when(w < extra)
            def _():
                pltpu.async_copy(ones_v, acc.at[dstv.at[base, 0]], sem,
                                 add=True).wait()
        descs = []
        for r in range(base):
            if r >= 12:
                descs[r - 12].wait()
            descs.append(
                pltpu.async_copy(ones_v, acc.at[dstv.at[r, 0]], sem, add=True)
            )
        for d in descs[-12:]:
            d.wait()
        plsc.subcore_barrier()
        pltpu.sync_copy(
            acc.at[pl.ds(r0, RPT)], out_hbm.at[c, pl.ds(r0, RPT)]
        )

    return deg_kernel(e2, ones_r, z16)


def _conv1_pass(h0, degp2, e2, z32, e_rows):
    base, extra, chunks = _edge_geometry(e_rows)
    chunk_max = max(chunks) * 128

    @functools.partial(
        pl.kernel,
        out_type=(
            jax.ShapeDtypeStruct((NC, NPAD, BN), jnp.float32),
            jax.ShapeDtypeStruct((NPAD, 16), jnp.float32),
        ),
        mesh=_sc_mesh(),
        scratch_types=[
            pltpu.VMEM((base + 1, 1, 128), jnp.int32),
            pltpu.VMEM((base + 1, 1, 128), jnp.int32),
            pltpu.VMEM((chunk_max, BN), jnp.float32),
            pltpu.VMEM((chunk_max, BN), jnp.float32),
            pltpu.VMEM((RPT, 16), jnp.float32),
            pltpu.SemaphoreType.DMA,
            pltpu.SemaphoreType.DMA,
            pltpu.SemaphoreType.DMA,
            pltpu.SemaphoreType.DMA,
            pltpu.VMEM_SHARED((NPAD, BN), jnp.float32),
            pltpu.HBM((NPAD, BN), jnp.float32),
        ],
        compiler_params=_SC_PARAMS,
    )
    def conv1_kernel(h0_hbm, degp_hbm, e_hbm, z_hbm,
                     out_hbm, dis_hbm,
                     srcv, dstv, rows0, rows1, disv,
                     gsem0, gsem1, ssem0, ssem1, acc, tbl):
        c = lax.axis_index("c")
        s = lax.axis_index("s")
        w = c * NS + s
        r0 = s * RPT
        _stage_edges(e_hbm, None, base, extra, e_rows, srcv, dstv, w)
        pltpu.sync_copy(degp_hbm.at[0, pl.ds(s * HPT, HPT)],
                        rows1.at[pl.ds(0, HPT)])
        pltpu.sync_copy(degp_hbm.at[1, pl.ds(s * HPT, HPT)],
                        rows1.at[pl.ds(HPT, HPT)])
        pltpu.sync_copy(h0_hbm.at[pl.ds(r0, RPT)], rows0.at[pl.ds(0, RPT)])

        def pro(q, carry):
            for u in range(2):
                p = 2 * q + u
                for h in range(2):
                    d = (rows1[p, pl.ds(16 * h, 16)]
                         + rows1[HPT + p, pl.ds(16 * h, 16)])
                    y = _vrsqrt(jnp.maximum(d, 1.0))
                    r = 2 * p + h
                    disv[r] = y
                    rows0[r, pl.ds(0, 16)] = rows0[r, pl.ds(0, 16)] * y
                    rows0[r, pl.ds(16, 16)] = rows0[r, pl.ds(16, 16)] * y
            return carry

        lax.fori_loop(0, HPT // 2, pro, 0)
        pltpu.sync_copy(rows0.at[pl.ds(0, RPT)], tbl.at[pl.ds(r0, RPT)])
        pltpu.sync_copy(disv, dis_hbm.at[pl.ds(r0, RPT)])

        @pl.when(c == 0)
        def _():
            pltpu.sync_copy(rows0.at[pl.ds(0, RPT)], acc.at[pl.ds(r0, RPT)])

        @pl.when(c != 0)
        def _():
            pltpu.sync_copy(z_hbm, acc.at[pl.ds(r0, RPT)])

        plsc.subcore_barrier()
        _extra_edge(tbl, srcv, dstv, rows1, gsem1, acc, base, extra, w)
        _edge_loop(tbl, srcv, dstv, (rows0, rows1),
                   (gsem0, gsem1), (ssem0, ssem1), acc, chunks)
        plsc.subcore_barrier()
        pltpu.sync_copy(
            acc.at[pl.ds(r0, RPT)], out_hbm.at[c, pl.ds(r0, RPT)]
        )

    return conv1_kernel(h0, degp2, e2, z32)


def _conv2_pass(m1p, dis16, b_row, e2, z32, e_rows):
    base, extra, chunks = _edge_geometry(e_rows)
    chunk_max = max(chunks) * 128

    @functools.partial(
        pl.kernel,
        out_type=jax.ShapeDtypeStruct((NC, NPAD, BN), jnp.float32),
        mesh=_sc_mesh(),
        scratch_types=[
            pltpu.VMEM((base + 1, 1, 128), jnp.int32),
            pltpu.VMEM((base + 1, 1, 128), jnp.int32),
            pltpu.VMEM((chunk_max, BN), jnp.float32),
            pltpu.VMEM((chunk_max, BN), jnp.float32),
            pltpu.VMEM((RPT, 16), jnp.float32),
            pltpu.VMEM((1, BN), jnp.float32),
            pltpu.SemaphoreType.DMA,
            pltpu.SemaphoreType.DMA,
            pltpu.SemaphoreType.DMA,
            pltpu.SemaphoreType.DMA,
            pltpu.VMEM_SHARED((NPAD, BN), jnp.float32),
            pltpu.HBM((NPAD, BN), jnp.float32),
        ],
        compiler_params=_SC_PARAMS,
    )
    def conv2_kernel(m1p_hbm, dis_hbm, b_hbm, e_hbm, z_hbm,
                     out_hbm,
                     srcv, dstv, rows0, rows1, disv, bv,
                     gsem0, gsem1, ssem0, ssem1, acc, tbl):
        c = lax.axis_index("c")
        s = lax.axis_index("s")
        w = c * NS + s
        r0 = s * RPT
        _stage_edges(e_hbm, None, base, extra, e_rows, srcv, dstv, w)
        pltpu.sync_copy(m1p_hbm.at[0, pl.ds(r0, RPT)], rows0.at[pl.ds(0, RPT)])
        pltpu.sync_copy(m1p_hbm.at[1, pl.ds(r0, RPT)], rows1.at[pl.ds(0, RPT)])
        pltpu.sync_copy(dis_hbm.at[pl.ds(r0, RPT)], disv)
        pltpu.sync_copy(b_hbm, bv)

        def pro(q, carry):
            for u in range(4):
                r = 4 * q + u
                y = disv[r]
                for h in range(2):
                    v = (rows0[r, pl.ds(16 * h, 16)]
                         + rows1[r, pl.ds(16 * h, 16)]) * y
                    v = jnp.maximum(v + bv[0, pl.ds(16 * h, 16)], 0.0) * y
                    rows0[r, pl.ds(16 * h, 16)] = v
            return carry

        lax.fori_loop(0, RPT // 4, pro, 0)
        pltpu.sync_copy(rows0.at[pl.ds(0, RPT)], tbl.at[pl.ds(r0, RPT)])

        @pl.when(c == 0)
        def _():
            pltpu.sync_copy(rows0.at[pl.ds(0, RPT)], acc.at[pl.ds(r0, RPT)])

        @pl.when(c != 0)
        def _():
            pltpu.sync_copy(z_hbm, acc.at[pl.ds(r0, RPT)])

        plsc.subcore_barrier()
        _extra_edge(tbl, srcv, dstv, rows1, gsem1, acc, base, extra, w)
        _edge_loop(tbl, srcv, dstv, (rows0, rows1),
                   (gsem0, gsem1), (ssem0, ssem1), acc, chunks)
        plsc.subcore_barrier()
        pltpu.sync_copy(
            acc.at[pl.ds(r0, RPT)], out_hbm.at[c, pl.ds(r0, RPT)]
        )

    return conv2_kernel(m1p, dis16, b_row, e2, z32)


def _tc_down(x, w_down):
    def body(x_ref, wd_ref, h0_ref):
        h0_ref[...] = lax.dot_general(
            x_ref[...], wd_ref[...], (((1,), (1,)), ((), ())),
            preferred_element_type=jnp.float32,
        )

    return pl.pallas_call(
        body,
        grid=(N // RB,),
        in_specs=[
            pl.BlockSpec((RB, H), lambda i: (i, 0)),
            pl.BlockSpec((BN, H), lambda i: (0, 0)),
        ],
        out_specs=pl.BlockSpec((RB, BN), lambda i: (i, 0)),
        out_shape=jax.ShapeDtypeStruct((NPAD, BN), jnp.float32),
    )(x, w_down)


def _tc_up(m2p, dis16, w_up, b_up_row, x):
    def body(m2p_ref, dis_ref, wu_ref, b_ref, x_ref, out_ref):
        m2 = (m2p_ref[0] + m2p_ref[1]) * dis_ref[...][:, :1]
        y = lax.dot_general(
            m2, wu_ref[...], (((1,), (1,)), ((), ())),
            preferred_element_type=jnp.float32,
        )
        out_ref[...] = y + b_ref[...] + x_ref[...]

    return pl.pallas_call(
        body,
        grid=(N // RB,),
        in_specs=[
            pl.BlockSpec((NC, RB, BN), lambda i: (0, i, 0)),
            pl.BlockSpec((RB, 16), lambda i: (i, 0)),
            pl.BlockSpec((H, BN), lambda i: (0, 0)),
            pl.BlockSpec((1, H), lambda i: (0, 0)),
            pl.BlockSpec((RB, H), lambda i: (i, 0)),
        ],
        out_specs=pl.BlockSpec((RB, H), lambda i: (i, 0)),
        out_shape=jax.ShapeDtypeStruct((N, H), jnp.float32),
    )(m2p, dis16, w_up, b_up_row, x)


def kernel(x, edge_index, W_down, b_down, W_up, b_up):
    f32 = jnp.float32
    e = edge_index.shape[1]
    if e % 128:
        npad_e = 128 - e % 128
        pad = jnp.stack([
            jnp.zeros((npad_e,), jnp.int32),
            jnp.full((npad_e,), DUMP, jnp.int32),
        ])
        edge_index = jnp.concatenate([edge_index, pad], axis=1)
        e += npad_e
    e_rows = e // 128
    e2 = edge_index.reshape(2, e_rows, 1, 128)
    z16 = jnp.zeros((RPT, 16), f32)
    z32 = jnp.zeros((RPT, BN), f32)
    ones_r = jnp.ones((RPT, 16), f32)

    h0 = _tc_down(x, W_down)
    degp = _deg_pass(e2, ones_r, z16, e_rows)
    m1p, dis16 = _conv1_pass(
        h0, degp.reshape(NC, NPAD // 2, 32), e2, z32, e_rows
    )
    m2p = _conv2_pass(m1p, dis16, b_down.reshape(1, BN), e2, z32, e_rows)
    return _tc_up(m2p, dis16, W_up, b_up.reshape(1, H), x)


# speedup vs baseline: 1.8867x; 1.0274x over previous
import functools

import jax
import jax.numpy as jnp
from jax import lax
from jax.experimental import pallas as pl
from jax.experimental.pallas import tpu as pltpu
from jax.experimental.pallas import tpu_sc as plsc

N = 10000
H = 128
BN = 32
NPAD = 10240
NC, NS = 2, 16
NW = NC * NS
G = 6
DUMP = N
RPT = NPAD // NS
HPT = RPT // 2
RB = 1000


def _sc_mesh():
    return plsc.VectorSubcoreMesh(
        core_axis_name="c", subcore_axis_name="s", num_cores=NC, num_subcores=NS
    )


_SC_PARAMS = pltpu.CompilerParams(
    use_tc_tiling_on_sc=False, needs_layout_passes=False
)


def _edge_geometry(e_rows):
    base = e_rows // NW
    extra = e_rows % NW
    chunks = [G] * (base // G)
    if base % G:
        chunks.append(base % G)
    return base, extra, chunks


def _stage_edges(e_hbm, row, base, extra, e_rows, srcv, dstv, w):
    pltpu.sync_copy(e_hbm.at[0, pl.ds(w * base, base)], srcv.at[pl.ds(0, base)])
    pltpu.sync_copy(e_hbm.at[1, pl.ds(w * base, base)], dstv.at[pl.ds(0, base)])
    del row
    if extra:
        off = e_rows - extra + lax.min(w, extra - 1)
        pltpu.sync_copy(e_hbm.at[0, pl.ds(off, 1)], srcv.at[pl.ds(base, 1)])
        pltpu.sync_copy(e_hbm.at[1, pl.ds(off, 1)], dstv.at[pl.ds(base, 1)])


def _edge_loop(tbl, srcv, dstv, rows, gsem, ssem, acc, chunks):
    starts = [0]
    for g in chunks:
        starts.append(starts[-1] + g)
    nch = len(chunks)

    def fire_gather(k):
        b = k & 1
        return [
            pltpu.async_copy(
                tbl.at[srcv.at[starts[k] + g, 0]],
                rows[b].at[pl.ds(g * 128, 128)],
                gsem[b],
            )
            for g in range(chunks[k])
        ]

    def fire_scatter(k):
        b = k & 1
        return [
            pltpu.async_copy(
                rows[b].at[pl.ds(g * 128, 128)],
                acc.at[dstv.at[starts[k] + g, 0]],
                ssem[b],
                add=True,
            )
            for g in range(chunks[k])
        ]

    def drain_fire(k):
        b = k & 1
        out = []
        for g, d in enumerate(gd[k]):
            d.wait()
            out.append(
                pltpu.async_copy(
                    rows[b].at[pl.ds(g * 128, 128)],
                    acc.at[dstv.at[starts[k] + g, 0]],
                    ssem[b],
                    add=True,
                )
            )
        return out

    del fire_scatter
    gd = {0: fire_gather(0)}
    sd = {}
    for k in range(nch):
        if k + 1 < nch:
            if k - 1 >= 0:
                for d in sd[k - 1]:
                    d.wait()
            gd[k + 1] = fire_gather(k + 1)
        sd[k] = drain_fire(k)
    for k in range(max(0, nch - 2), nch):
        for d in sd[k]:
            d.wait()


def _extra_edge(tbl, srcv, dstv, buf, sem, acc, base, extra, w):
    if not extra:
        return

    @pl.when(w < extra)
    def _():
        pltpu.async_copy(
            tbl.at[srcv.at[base, 0]], buf.at[pl.ds(0, 128)], sem
        ).wait()
        pltpu.async_copy(
            buf.at[pl.ds(0, 128)], acc.at[dstv.at[base, 0]], sem, add=True
        ).wait()


def _deg_pass(e2, ones_r, z16, e_rows):
    base, extra, chunks = _edge_geometry(e_rows)

    @functools.partial(
        pl.kernel,
        out_type=jax.ShapeDtypeStruct((NC, NPAD, 16), jnp.float32),
        mesh=_sc_mesh(),
        scratch_types=[
            pltpu.VMEM((base + 1, 1, 128), jnp.int32),
            pltpu.VMEM((128, 16), jnp.float32),
            pltpu.VMEM_SHARED((NPAD, 16), jnp.float32),
            pltpu.SemaphoreType.DMA,
        ],
        compiler_params=_SC_PARAMS,
    )
    def deg_kernel(e_hbm, ones_hbm, z_hbm, out_hbm, dstv, ones_v, acc, sem):
        c = lax.axis_index("c")
        s = lax.axis_index("s")
        w = c * NS + s
        r0 = s * RPT

        @pl.when(c == 0)
        def _():
            pltpu.sync_copy(ones_hbm, acc.at[pl.ds(r0, RPT)])

        @pl.when(c != 0)
        def _():
            pltpu.sync_copy(z_hbm, acc.at[pl.ds(r0, RPT)])

        pltpu.sync_copy(ones_hbm.at[pl.ds(0, 128)], ones_v)
        pltpu.sync_copy(e_hbm.at[1, pl.ds(w * base, base)],
                        dstv.at[pl.ds(0, base)])
        if extra:
            off = e_rows - extra + lax.min(w, extra - 1)
            pltpu.sync_copy(e_hbm.at[1, pl.ds(off, 1)], dstv.at[pl.ds(base, 1)])
        plsc.subcore_barrier()
        if extra:
            @pl.when(w < extra)
            def _():
                pltpu.async_copy(ones_v, acc.at[dstv.at[base, 0]], sem,
                                 add=True).wait()
        descs = []
        for r in range(base):
            if r >= 12:
                descs[r - 12].wait()
            descs.append(
                pltpu.async_copy(ones_v, acc.at[dstv.at[r, 0]], sem, add=True)
            )
        for d in descs[-12:]:
            d.wait()
        plsc.subcore_barrier()
        pltpu.sync_copy(
            acc.at[pl.ds(r0, RPT)], out_hbm.at[c, pl.ds(r0, RPT)]
        )

    return deg_kernel(e2, ones_r, z16)


def _conv1_pass(h0s, e2, z32, e_rows):
    base, extra, chunks = _edge_geometry(e_rows)
    chunk_max = max(chunks) * 128

    @functools.partial(
        pl.kernel,
        out_type=jax.ShapeDtypeStruct((NC, NPAD, BN), jnp.float32),
        mesh=_sc_mesh(),
        scratch_types=[
            pltpu.VMEM((base + 1, 1, 128), jnp.int32),
            pltpu.VMEM((base + 1, 1, 128), jnp.int32),
            pltpu.VMEM((chunk_max, BN), jnp.float32),
            pltpu.VMEM((chunk_max, BN), jnp.float32),
            pltpu.SemaphoreType.DMA,
            pltpu.SemaphoreType.DMA,
            pltpu.SemaphoreType.DMA,
            pltpu.SemaphoreType.DMA,
            pltpu.VMEM_SHARED((NPAD, BN), jnp.float32),
        ],
        compiler_params=_SC_PARAMS,
    )
    def conv1_kernel(tbl_hbm, e_hbm, z_hbm, out_hbm,
                     srcv, dstv, rows0, rows1,
                     gsem0, gsem1, ssem0, ssem1, acc):
        c = lax.axis_index("c")
        s = lax.axis_index("s")
        w = c * NS + s
        r0 = s * RPT
        _stage_edges(e_hbm, None, base, extra, e_rows, srcv, dstv, w)

        @pl.when(c == 0)
        def _():
            pltpu.sync_copy(tbl_hbm.at[pl.ds(r0, RPT)], acc.at[pl.ds(r0, RPT)])

        @pl.when(c != 0)
        def _():
            pltpu.sync_copy(z_hbm, acc.at[pl.ds(r0, RPT)])

        plsc.subcore_barrier()
        _extra_edge(tbl_hbm, srcv, dstv, rows1, gsem1, acc, base, extra, w)
        _edge_loop(tbl_hbm, srcv, dstv, (rows0, rows1),
                   (gsem0, gsem1), (ssem0, ssem1), acc, chunks)
        plsc.subcore_barrier()
        pltpu.sync_copy(
            acc.at[pl.ds(r0, RPT)], out_hbm.at[c, pl.ds(r0, RPT)]
        )

    return conv1_kernel(h0s, e2, z32)


def _tc_scale(degp, h0):
    def body(degp_ref, h0_ref, h0s_ref, dis_ref):
        deg = degp_ref[0] + degp_ref[1]
        dis = lax.rsqrt(jnp.maximum(deg, 1.0))
        dis_ref[...] = dis
        h0s_ref[...] = h0_ref[...] * dis[:, :1]

    rb = 1024
    return pl.pallas_call(
        body,
        grid=(NPAD // rb,),
        in_specs=[
            pl.BlockSpec((NC, rb, 16), lambda i: (0, i, 0)),
            pl.BlockSpec((rb, BN), lambda i: (i, 0)),
        ],
        out_specs=[
            pl.BlockSpec((rb, BN), lambda i: (i, 0)),
            pl.BlockSpec((rb, 16), lambda i: (i, 0)),
        ],
        out_shape=[
            jax.ShapeDtypeStruct((NPAD, BN), jnp.float32),
            jax.ShapeDtypeStruct((NPAD, 16), jnp.float32),
        ],
    )(degp, h0)


def _conv2_pass(m1p, dis16, b_row, e2, z32, e_rows):
    base, extra, chunks = _edge_geometry(e_rows)
    chunk_max = max(chunks) * 128

    @functools.partial(
        pl.kernel,
        out_type=jax.ShapeDtypeStruct((NC, NPAD, BN), jnp.float32),
        mesh=_sc_mesh(),
        scratch_types=[
            pltpu.VMEM((base + 1, 1, 128), jnp.int32),
            pltpu.VMEM((base + 1, 1, 128), jnp.int32),
            pltpu.VMEM((chunk_max, BN), jnp.float32),
            pltpu.VMEM((chunk_max, BN), jnp.float32),
            pltpu.VMEM((RPT, 16), jnp.float32),
            pltpu.VMEM((1, BN), jnp.float32),
            pltpu.SemaphoreType.DMA,
            pltpu.SemaphoreType.DMA,
            pltpu.SemaphoreType.DMA,
            pltpu.SemaphoreType.DMA,
            pltpu.VMEM_SHARED((NPAD, BN), jnp.float32),
            pltpu.HBM((NPAD, BN), jnp.float32),
        ],
        compiler_params=_SC_PARAMS,
    )
    def conv2_kernel(m1p_hbm, dis_hbm, b_hbm, e_hbm, z_hbm,
                     out_hbm,
                     srcv, dstv, rows0, rows1, disv, bv,
                     gsem0, gsem1, ssem0, ssem1, acc, tbl):
        c = lax.axis_index("c")
        s = lax.axis_index("s")
        w = c * NS + s
        r0 = s * RPT
        _stage_edges(e_hbm, None, base, extra, e_rows, srcv, dstv, w)
        pltpu.sync_copy(m1p_hbm.at[0, pl.ds(r0, RPT)], rows0.at[pl.ds(0, RPT)])
        pltpu.sync_copy(m1p_hbm.at[1, pl.ds(r0, RPT)], rows1.at[pl.ds(0, RPT)])
        pltpu.sync_copy(dis_hbm.at[pl.ds(r0, RPT)], disv)
        pltpu.sync_copy(b_hbm, bv)

        def pro(q, carry):
            for u in range(4):
                r = 4 * q + u
                y = disv[r]
                for h in range(2):
                    v = (rows0[r, pl.ds(16 * h, 16)]
                         + rows1[r, pl.ds(16 * h, 16)]) * y
                    v = jnp.maximum(v + bv[0, pl.ds(16 * h, 16)], 0.0) * y
                    rows0[r, pl.ds(16 * h, 16)] = v
            return carry

        lax.fori_loop(0, RPT // 4, pro, 0)
        pltpu.sync_copy(rows0.at[pl.ds(0, RPT)], tbl.at[pl.ds(r0, RPT)])

        @pl.when(c == 0)
        def _():
            pltpu.sync_copy(rows0.at[pl.ds(0, RPT)], acc.at[pl.ds(r0, RPT)])

        @pl.when(c != 0)
        def _():
            pltpu.sync_copy(z_hbm, acc.at[pl.ds(r0, RPT)])

        plsc.subcore_barrier()
        _extra_edge(tbl, srcv, dstv, rows1, gsem1, acc, base, extra, w)
        _edge_loop(tbl, srcv, dstv, (rows0, rows1),
                   (gsem0, gsem1), (ssem0, ssem1), acc, chunks)
        plsc.subcore_barrier()
        pltpu.sync_copy(
            acc.at[pl.ds(r0, RPT)], out_hbm.at[c, pl.ds(r0, RPT)]
        )

    return conv2_kernel(m1p, dis16, b_row, e2, z32)


def _tc_down(x, w_down):
    def body(x_ref, wd_ref, h0_ref):
        h0_ref[...] = lax.dot_general(
            x_ref[...], wd_ref[...], (((1,), (1,)), ((), ())),
            preferred_element_type=jnp.float32,
        )

    return pl.pallas_call(
        body,
        grid=(N // RB,),
        in_specs=[
            pl.BlockSpec((RB, H), lambda i: (i, 0)),
            pl.BlockSpec((BN, H), lambda i: (0, 0)),
        ],
        out_specs=pl.BlockSpec((RB, BN), lambda i: (i, 0)),
        out_shape=jax.ShapeDtypeStruct((NPAD, BN), jnp.float32),
    )(x, w_down)


def _tc_up(m2p, dis16, w_up, b_up_row, x):
    def body(m2p_ref, dis_ref, wu_ref, b_ref, x_ref, out_ref):
        m2 = (m2p_ref[0] + m2p_ref[1]) * dis_ref[...][:, :1]
        y = lax.dot_general(
            m2, wu_ref[...], (((1,), (1,)), ((), ())),
            preferred_element_type=jnp.float32,
        )
        out_ref[...] = y + b_ref[...] + x_ref[...]

    return pl.pallas_call(
        body,
        grid=(N // RB,),
        in_specs=[
            pl.BlockSpec((NC, RB, BN), lambda i: (0, i, 0)),
            pl.BlockSpec((RB, 16), lambda i: (i, 0)),
            pl.BlockSpec((H, BN), lambda i: (0, 0)),
            pl.BlockSpec((1, H), lambda i: (0, 0)),
            pl.BlockSpec((RB, H), lambda i: (i, 0)),
        ],
        out_specs=pl.BlockSpec((RB, H), lambda i: (i, 0)),
        out_shape=jax.ShapeDtypeStruct((N, H), jnp.float32),
    )(m2p, dis16, w_up, b_up_row, x)


def kernel(x, edge_index, W_down, b_down, W_up, b_up):
    f32 = jnp.float32
    e = edge_index.shape[1]
    if e % 128:
        npad_e = 128 - e % 128
        pad = jnp.stack([
            jnp.zeros((npad_e,), jnp.int32),
            jnp.full((npad_e,), DUMP, jnp.int32),
        ])
        edge_index = jnp.concatenate([edge_index, pad], axis=1)
        e += npad_e
    e_rows = e // 128
    e2 = edge_index.reshape(2, e_rows, 1, 128)
    z16 = jnp.zeros((RPT, 16), f32)
    z32 = jnp.zeros((RPT, BN), f32)
    ones_r = jnp.ones((RPT, 16), f32)

    h0 = _tc_down(x, W_down)
    degp = _deg_pass(e2, ones_r, z16, e_rows)
    h0s, dis16 = _tc_scale(degp, h0)
    m1p = _conv1_pass(h0s, e2, z32, e_rows)
    m2p = _conv2_pass(m1p, dis16, b_down.reshape(1, BN), e2, z32, e_rows)
    return _tc_up(m2p, dis16, W_up, b_up.reshape(1, H), x)
